# Initial kernel scaffold; baseline (speedup 1.0000x reference)
import jax
import jax.numpy as jnp
from jax.experimental import pallas as pl


def kernel(seed_nodes, t_eval_array, fanouts, delta_t, neighbor_indices, neighbor_times, node_features, W1, W2):
    raise NotImplementedError("write your pallas kernel here")


# speedup vs baseline: 12.0797x; 12.0797x over previous
import functools

import jax
import jax.numpy as jnp
from jax import lax
from jax.experimental import pallas as pl
from jax.experimental.pallas import tpu as pltpu
from jax.experimental.pallas import tpu_sc as plsc

_NC = 2
_NS = 16
_L = 16
_NW = _NC * _NS

_B = 4096
_DEG = 32
_D = 128
_BPW = _B // _NW
_G = 4


def _sc_body(seed_hbm, tadj_hbm, nbridx_hbm, nbrt_hbm, feat_hbm,
             s_out, fs_out, tg_out,
             seeds_v, tadj_v, nbr_v, tms_v, idxp_v, fs_v, s_v, buf_v, sem):
    sent = jnp.int32(feat_hbm.shape[0] - 1)
    wid = lax.axis_index("s") * _NC + lax.axis_index("c")
    base = wid * _BPW

    pltpu.sync_copy(seed_hbm.at[pl.ds(base, _BPW)], seeds_v)
    pltpu.sync_copy(tadj_hbm.at[pl.ds(base, _BPW)], tadj_v)

    pltpu.async_copy(nbridx_hbm.at[seeds_v], nbr_v, sem).wait()
    pltpu.async_copy(nbrt_hbm.at[seeds_v], tms_v, sem).wait()
    pltpu.async_copy(feat_hbm.at[seeds_v], fs_v, sem).wait()
    pltpu.sync_copy(tms_v, tg_out.at[pl.ds(base, _BPW)])
    pltpu.sync_copy(fs_v, fs_out.at[pl.ds(base, _BPW)])

    def mask_body(gi, carry):
        tval = tadj_v[pl.ds(gi * _L, _L)]
        for k in range(_L):
            i = gi * _L + k
            tv = jnp.full((_L,), tval[k], jnp.float32)
            for c in range(_DEG // _L):
                tm = tms_v[i, pl.ds(c * _L, _L)]
                nb = nbr_v[i, pl.ds(c * _L, _L)]
                idxp_v[pl.ds(i * _DEG + c * _L, _L)] = jnp.where(tm <= tv, nb, sent)
        return carry

    lax.fori_loop(0, _BPW // _L, mask_body, 0)

    def gather_body(g, carry):
        cp = pltpu.async_copy(
            feat_hbm.at[idxp_v.at[pl.ds(g * _G * _DEG, _G * _DEG)]], buf_v, sem)
        cp.wait()
        for k in range(_G):
            for c in range(_D // _L):
                acc = buf_v[k * _DEG, pl.ds(c * _L, _L)]
                for j in range(1, _DEG):
                    acc = acc + buf_v[k * _DEG + j, pl.ds(c * _L, _L)]
                s_v[g * _G + k, pl.ds(c * _L, _L)] = acc
        return carry

    lax.fori_loop(0, _BPW // _G, gather_body, 0)
    pltpu.sync_copy(s_v, s_out.at[pl.ds(base, _BPW)])


def _sc_gather(seed_nodes, t_adj, neighbor_indices, neighbor_times, feat_ext):
    mesh = plsc.VectorSubcoreMesh(core_axis_name="c", subcore_axis_name="s")
    fn = pl.kernel(
        _sc_body,
        out_type=(
            jax.ShapeDtypeStruct((_B, _D), jnp.float32),
            jax.ShapeDtypeStruct((_B, _D), jnp.float32),
            jax.ShapeDtypeStruct((_B, _DEG), jnp.float32),
        ),
        mesh=mesh,
        compiler_params=pltpu.CompilerParams(use_tc_tiling_on_sc=False),
        scratch_types=[
            pltpu.VMEM((_BPW,), jnp.int32),
            pltpu.VMEM((_BPW,), jnp.float32),
            pltpu.VMEM((_BPW, _DEG), jnp.int32),
            pltpu.VMEM((_BPW, _DEG), jnp.float32),
            pltpu.VMEM((_BPW * _DEG,), jnp.int32),
            pltpu.VMEM((_BPW, _D), jnp.float32),
            pltpu.VMEM((_BPW, _D), jnp.float32),
            pltpu.VMEM((_G * _DEG, _D), jnp.float32),
            pltpu.SemaphoreType.DMA,
        ],
    )
    return fn(seed_nodes, t_adj, neighbor_indices, neighbor_times, feat_ext)


def _tc_head_body(s_ref, fs_ref, tg_ref, ta_ref, w1_ref, w2_ref, o_ref):
    m = (tg_ref[...] <= ta_ref[...][:, None]).astype(jnp.float32)
    dcnt = jnp.sum(m, axis=1)
    pos = dcnt > 0.0
    dinv = jnp.where(pos, lax.rsqrt(jnp.maximum(dcnt, 1e-12)), 0.0)
    agg = jnp.where(pos[:, None],
                    0.5 * fs_ref[...] + 0.5 * dinv[:, None] * s_ref[...],
                    0.0)
    h = jnp.maximum(
        jnp.dot(agg, w1_ref[...], preferred_element_type=jnp.float32), 0.0)
    o_ref[...] = jnp.dot(h, w2_ref[...], preferred_element_type=jnp.float32)


def _tc_head(s, fs, tg, t_adj, W1, W2):
    blk = 1024
    ncls = W2.shape[1]
    return pl.pallas_call(
        _tc_head_body,
        grid=(_B // blk,),
        in_specs=[
            pl.BlockSpec((blk, _D), lambda i: (i, 0)),
            pl.BlockSpec((blk, _D), lambda i: (i, 0)),
            pl.BlockSpec((blk, _DEG), lambda i: (i, 0)),
            pl.BlockSpec((blk,), lambda i: (i,)),
            pl.BlockSpec((_D, W1.shape[1]), lambda i: (0, 0)),
            pl.BlockSpec((W2.shape[0], ncls), lambda i: (0, 0)),
        ],
        out_specs=pl.BlockSpec((blk, ncls), lambda i: (i, 0)),
        out_shape=jax.ShapeDtypeStruct((_B, ncls), jnp.float32),
    )(s, fs, tg, t_adj, W1, W2)


def kernel(seed_nodes, t_eval_array, fanouts, delta_t, neighbor_indices,
           neighbor_times, node_features, W1, W2):
    del fanouts
    t_adj = (t_eval_array + delta_t).astype(jnp.float32)
    feat_ext = jnp.concatenate(
        [node_features, jnp.zeros((1, node_features.shape[1]),
                                  node_features.dtype)], axis=0)
    s, fs, tg = _sc_gather(seed_nodes.astype(jnp.int32), t_adj,
                           neighbor_indices.astype(jnp.int32),
                           neighbor_times, feat_ext)
    return _tc_head(s, fs, tg, t_adj, W1, W2)


# speedup vs baseline: 14.3355x; 1.1867x over previous
import jax
import jax.numpy as jnp
from jax import lax
from jax.experimental import pallas as pl
from jax.experimental.pallas import tpu as pltpu
from jax.experimental.pallas import tpu_sc as plsc

_NC = 2
_NS = 16
_L = 16
_NW = _NC * _NS

_B = 4096
_DEG = 32
_D = 128
_BPW = _B // _NW
_G = 4
_NGRP = _BPW // _G


def _sc_body(seed_hbm, tadj_hbm, nbridx_hbm, nbrt_hbm, feat_hbm,
             s_out, fs_out, tg_out,
             seeds_v, tadj_v, nbr_v, tms_v, idxp_v, fs_v, s_v, buf_v,
             sem_a, sem_b, sem_c, sem_r0, sem_r1):
    wid = lax.axis_index("s") * _NC + lax.axis_index("c")
    base = wid * _BPW

    pltpu.sync_copy(seed_hbm.at[pl.ds(base, _BPW)], seeds_v)
    pltpu.sync_copy(tadj_hbm.at[pl.ds(base, _BPW)], tadj_v)

    nbr_cp = pltpu.async_copy(nbridx_hbm.at[seeds_v], nbr_v, sem_a)
    tms_cp = pltpu.async_copy(nbrt_hbm.at[seeds_v], tms_v, sem_b)
    fs_cp = pltpu.async_copy(feat_hbm.at[seeds_v], fs_v, sem_c)
    nbr_cp.wait()
    tms_cp.wait()

    def mask_body(gi, carry):
        tval = tadj_v[pl.ds(gi * _L, _L)]
        for k in range(_L):
            i = gi * _L + k
            tv = jnp.full((_L,), tval[k], jnp.float32)
            for c in range(_DEG // _L):
                tm = tms_v[i, pl.ds(c * _L, _L)]
                nb = nbr_v[i, pl.ds(c * _L, _L)]
                idxp_v[pl.ds(i * _DEG + c * _L, _L)] = jnp.where(
                    tm <= tv, nb, jnp.int32(0))
        return carry

    lax.fori_loop(0, _BPW // _L, mask_body, 0)

    ring = (sem_r0, sem_r1)

    def gather_desc(g, b):
        return pltpu.make_async_copy(
            feat_hbm.at[idxp_v.at[pl.ds(g * _G * _DEG, _G * _DEG)]],
            buf_v.at[b], ring[b])

    gather_desc(0, 0).start()

    def ring_body(go, carry):
        for b in range(2):
            g = go * 2 + b

            @pl.when(g + 1 < _NGRP)
            def _():
                gather_desc(g + 1, 1 - b).start()

            gather_desc(g, b).wait()
            for k in range(_G):
                for c in range(_D // _L):
                    acc = buf_v[b, k * _DEG, pl.ds(c * _L, _L)]
                    for j in range(1, _DEG):
                        acc = acc + buf_v[b, k * _DEG + j, pl.ds(c * _L, _L)]
                    s_v[g * _G + k, pl.ds(c * _L, _L)] = acc
        return carry

    lax.fori_loop(0, _NGRP // 2, ring_body, 0)

    pltpu.sync_copy(s_v, s_out.at[pl.ds(base, _BPW)])
    pltpu.sync_copy(tms_v, tg_out.at[pl.ds(base, _BPW)])
    fs_cp.wait()
    pltpu.sync_copy(fs_v, fs_out.at[pl.ds(base, _BPW)])


def _sc_gather(seed_nodes, t_adj, neighbor_indices, neighbor_times, feat):
    mesh = plsc.VectorSubcoreMesh(core_axis_name="c", subcore_axis_name="s")
    fn = pl.kernel(
        _sc_body,
        out_type=(
            jax.ShapeDtypeStruct((_B, _D), jnp.float32),
            jax.ShapeDtypeStruct((_B, _D), jnp.float32),
            jax.ShapeDtypeStruct((_B, _DEG), jnp.float32),
        ),
        mesh=mesh,
        compiler_params=pltpu.CompilerParams(use_tc_tiling_on_sc=False),
        scratch_types=[
            pltpu.VMEM((_BPW,), jnp.int32),
            pltpu.VMEM((_BPW,), jnp.float32),
            pltpu.VMEM((_BPW, _DEG), jnp.int32),
            pltpu.VMEM((_BPW, _DEG), jnp.float32),
            pltpu.VMEM((_BPW * _DEG,), jnp.int32),
            pltpu.VMEM((_BPW, _D), jnp.float32),
            pltpu.VMEM((_BPW, _D), jnp.float32),
            pltpu.VMEM((2, _G * _DEG, _D), jnp.float32),
            pltpu.SemaphoreType.DMA,
            pltpu.SemaphoreType.DMA,
            pltpu.SemaphoreType.DMA,
            pltpu.SemaphoreType.DMA,
            pltpu.SemaphoreType.DMA,
        ],
    )
    return fn(seed_nodes, t_adj, neighbor_indices, neighbor_times, feat)


def _tc_head_body(s_ref, fs_ref, tg_ref, ta_ref, f0_ref, w1_ref, w2_ref, o_ref):
    m = (tg_ref[...] <= ta_ref[...][:, None]).astype(jnp.float32)
    dcnt = jnp.sum(m, axis=1)
    pos = dcnt > 0.0
    dinv = jnp.where(pos, lax.rsqrt(jnp.maximum(dcnt, 1e-12)), 0.0)
    s_corr = s_ref[...] - (jnp.float32(_DEG) - dcnt)[:, None] * f0_ref[...]
    agg = jnp.where(pos[:, None],
                    0.5 * fs_ref[...] + 0.5 * dinv[:, None] * s_corr,
                    0.0)
    h = jnp.maximum(
        jnp.dot(agg, w1_ref[...], preferred_element_type=jnp.float32), 0.0)
    o_ref[...] = jnp.dot(h, w2_ref[...], preferred_element_type=jnp.float32)


def _tc_head(s, fs, tg, t_adj, feat0, W1, W2):
    blk = 1024
    ncls = W2.shape[1]
    return pl.pallas_call(
        _tc_head_body,
        grid=(_B // blk,),
        in_specs=[
            pl.BlockSpec((blk, _D), lambda i: (i, 0)),
            pl.BlockSpec((blk, _D), lambda i: (i, 0)),
            pl.BlockSpec((blk, _DEG), lambda i: (i, 0)),
            pl.BlockSpec((blk,), lambda i: (i,)),
            pl.BlockSpec((1, _D), lambda i: (0, 0)),
            pl.BlockSpec((_D, W1.shape[1]), lambda i: (0, 0)),
            pl.BlockSpec((W2.shape[0], ncls), lambda i: (0, 0)),
        ],
        out_specs=pl.BlockSpec((blk, ncls), lambda i: (i, 0)),
        out_shape=jax.ShapeDtypeStruct((_B, ncls), jnp.float32),
    )(s, fs, tg, t_adj, feat0, W1, W2)


def kernel(seed_nodes, t_eval_array, fanouts, delta_t, neighbor_indices,
           neighbor_times, node_features, W1, W2):
    del fanouts
    t_adj = (t_eval_array + delta_t).astype(jnp.float32)
    s, fs, tg = _sc_gather(seed_nodes.astype(jnp.int32), t_adj,
                           neighbor_indices.astype(jnp.int32),
                           neighbor_times, node_features)
    return _tc_head(s, fs, tg, t_adj, node_features[0:1], W1, W2)


# speedup vs baseline: 19.2732x; 1.3444x over previous
import jax
import jax.numpy as jnp
from jax import lax
from jax.experimental import pallas as pl
from jax.experimental.pallas import tpu as pltpu
from jax.experimental.pallas import tpu_sc as plsc

_NC = 2
_NS = 16
_L = 16
_NW = _NC * _NS

_B = 4096
_DEG = 32
_D = 128
_BPW = _B // _NW
_G = 4
_GPB = 1
_NGRP = _BPW // _G


def _sc_body(seed_hbm, tadj_hbm, nbridx_hbm, nbrt_hbm, feat_hbm,
             s_out, fs_out, tg_out,
             seeds_v, tadj_v, nbr_v, tms_v, idxp_v, fs_v, s_v, buf_v,
             sem_a, sem_b, sem_c, sem_r0, sem_r1):
    wid = lax.axis_index("s") * _NC + lax.axis_index("c")
    base = wid * _BPW

    pltpu.sync_copy(seed_hbm.at[pl.ds(base, _BPW)], seeds_v)
    pltpu.sync_copy(tadj_hbm.at[pl.ds(base, _BPW)], tadj_v)

    nbr_cp = pltpu.async_copy(nbridx_hbm.at[seeds_v], nbr_v, sem_a)
    tms_cp = pltpu.async_copy(nbrt_hbm.at[seeds_v], tms_v, sem_b)
    fs_cp = pltpu.async_copy(feat_hbm.at[seeds_v], fs_v, sem_c)
    nbr_cp.wait()
    tms_cp.wait()

    def mask_body(gi, carry):
        tval = tadj_v[pl.ds(gi * _L, _L)]
        for k in range(_L):
            i = gi * _L + k
            tv = jnp.full((_L,), tval[k], jnp.float32)
            for c in range(_DEG // _L):
                tm = tms_v[i, pl.ds(c * _L, _L)]
                nb = nbr_v[i, pl.ds(c * _L, _L)]
                idxp_v[pl.ds(i * _DEG + c * _L, _L)] = jnp.where(
                    tm <= tv, nb, jnp.int32(0))
        return carry

    lax.fori_loop(0, _BPW // _L, mask_body, 0)

    ring = (sem_r0, sem_r1)
    seeds_per_slot = _GPB * _G
    nslots = _BPW // seeds_per_slot

    def slot_descs(sl, b):
        return [
            pltpu.make_async_copy(
                feat_hbm.at[idxp_v.at[pl.ds((sl * _GPB + p) * _G * _DEG,
                                            _G * _DEG)]],
                buf_v.at[b].at[pl.ds(p * _G * _DEG, _G * _DEG)],
                ring[b])
            for p in range(_GPB)
        ]

    def slot_start(sl, b):
        for d in slot_descs(sl, b):
            d.start()

    def slot_wait(sl, b):
        for d in slot_descs(sl, b):
            d.wait()

    slot_start(0, 0)

    def ring_body(so, carry):
        for b in range(2):
            sl = so * 2 + b

            @pl.when(sl + 1 < nslots)
            def _():
                slot_start(sl + 1, 1 - b)

            slot_wait(sl, b)
            for k in range(seeds_per_slot):
                accs = [buf_v[b, k * _DEG, pl.ds(c * _L, _L)]
                        for c in range(_D // _L)]
                for j in range(1, _DEG):
                    for c in range(_D // _L):
                        accs[c] = accs[c] + buf_v[b, k * _DEG + j,
                                                  pl.ds(c * _L, _L)]
                for c in range(_D // _L):
                    s_v[sl * seeds_per_slot + k, pl.ds(c * _L, _L)] = accs[c]
        return carry

    lax.fori_loop(0, nslots // 2, ring_body, 0)

    pltpu.sync_copy(s_v, s_out.at[pl.ds(base, _BPW)])
    pltpu.sync_copy(tms_v, tg_out.at[pl.ds(base, _BPW)])
    fs_cp.wait()
    pltpu.sync_copy(fs_v, fs_out.at[pl.ds(base, _BPW)])


def _sc_gather(seed_nodes, t_adj, neighbor_indices, neighbor_times, feat):
    mesh = plsc.VectorSubcoreMesh(core_axis_name="c", subcore_axis_name="s")
    fn = pl.kernel(
        _sc_body,
        out_type=(
            jax.ShapeDtypeStruct((_B, _D), jnp.float32),
            jax.ShapeDtypeStruct((_B, _D), jnp.float32),
            jax.ShapeDtypeStruct((_B, _DEG), jnp.float32),
        ),
        mesh=mesh,
        compiler_params=pltpu.CompilerParams(use_tc_tiling_on_sc=False),
        scratch_types=[
            pltpu.VMEM((_BPW,), jnp.int32),
            pltpu.VMEM((_BPW,), jnp.float32),
            pltpu.VMEM((_BPW, _DEG), jnp.int32),
            pltpu.VMEM((_BPW, _DEG), jnp.float32),
            pltpu.VMEM((_BPW * _DEG,), jnp.int32),
            pltpu.VMEM((_BPW, _D), jnp.float32),
            pltpu.VMEM((_BPW, _D), jnp.float32),
            pltpu.VMEM((2, _GPB * _G * _DEG, _D), jnp.float32),
            pltpu.SemaphoreType.DMA,
            pltpu.SemaphoreType.DMA,
            pltpu.SemaphoreType.DMA,
            pltpu.SemaphoreType.DMA,
            pltpu.SemaphoreType.DMA,
        ],
    )
    return fn(seed_nodes, t_adj, neighbor_indices, neighbor_times, feat)


def _tc_head_body(s_ref, fs_ref, tg_ref, ta_ref, f0_ref, w1_ref, w2_ref, o_ref):
    m = (tg_ref[...] <= ta_ref[...][:, None]).astype(jnp.float32)
    dcnt = jnp.sum(m, axis=1)
    pos = dcnt > 0.0
    dinv = jnp.where(pos, lax.rsqrt(jnp.maximum(dcnt, 1e-12)), 0.0)
    s_corr = s_ref[...] - (jnp.float32(_DEG) - dcnt)[:, None] * f0_ref[...]
    agg = jnp.where(pos[:, None],
                    0.5 * fs_ref[...] + 0.5 * dinv[:, None] * s_corr,
                    0.0)
    h = jnp.maximum(
        jnp.dot(agg, w1_ref[...], preferred_element_type=jnp.float32), 0.0)
    o_ref[...] = jnp.dot(h, w2_ref[...], preferred_element_type=jnp.float32)


def _tc_head(s, fs, tg, t_adj, feat0, W1, W2):
    blk = 1024
    ncls = W2.shape[1]
    return pl.pallas_call(
        _tc_head_body,
        grid=(_B // blk,),
        in_specs=[
            pl.BlockSpec((blk, _D), lambda i: (i, 0)),
            pl.BlockSpec((blk, _D), lambda i: (i, 0)),
            pl.BlockSpec((blk, _DEG), lambda i: (i, 0)),
            pl.BlockSpec((blk,), lambda i: (i,)),
            pl.BlockSpec((1, _D), lambda i: (0, 0)),
            pl.BlockSpec((_D, W1.shape[1]), lambda i: (0, 0)),
            pl.BlockSpec((W2.shape[0], ncls), lambda i: (0, 0)),
        ],
        out_specs=pl.BlockSpec((blk, ncls), lambda i: (i, 0)),
        out_shape=jax.ShapeDtypeStruct((_B, ncls), jnp.float32),
    )(s, fs, tg, t_adj, feat0, W1, W2)


def kernel(seed_nodes, t_eval_array, fanouts, delta_t, neighbor_indices,
           neighbor_times, node_features, W1, W2):
    del fanouts
    t_adj = (t_eval_array + delta_t).astype(jnp.float32)
    s, fs, tg = _sc_gather(seed_nodes.astype(jnp.int32), t_adj,
                           neighbor_indices.astype(jnp.int32),
                           neighbor_times, node_features)
    return _tc_head(s, fs, tg, t_adj, node_features[0:1], W1, W2)


# speedup vs baseline: 19.6117x; 1.0176x over previous
import jax
import jax.numpy as jnp
from jax import lax
from jax.experimental import pallas as pl
from jax.experimental.pallas import tpu as pltpu
from jax.experimental.pallas import tpu_sc as plsc

_NC = 2
_NS = 16
_L = 16
_NW = _NC * _NS

_B = 4096
_DEG = 32
_D = 128
_BPW = _B // _NW
_G = 2
_RING = 4
_NSLOT = _BPW // _G


def _sc_body(seed_hbm, tadj_hbm, nbridx_hbm, nbrt_hbm, feat_hbm,
             s_out, fs_out, tg_out,
             seeds_v, tadj_v, nbr_v, tms_v, idxp_v, fs_v, s_v, buf_v,
             sem_a, sem_b, sem_c, sem_r0, sem_r1, sem_r2, sem_r3):
    wid = lax.axis_index("s") * _NC + lax.axis_index("c")
    base = wid * _BPW

    pltpu.sync_copy(seed_hbm.at[pl.ds(base, _BPW)], seeds_v)
    pltpu.sync_copy(tadj_hbm.at[pl.ds(base, _BPW)], tadj_v)

    nbr_cp = pltpu.async_copy(nbridx_hbm.at[seeds_v], nbr_v, sem_a)
    tms_cp = pltpu.async_copy(nbrt_hbm.at[seeds_v], tms_v, sem_b)
    fs_cp = pltpu.async_copy(feat_hbm.at[seeds_v], fs_v, sem_c)
    nbr_cp.wait()
    tms_cp.wait()

    def mask_body(gi, carry):
        tval = tadj_v[pl.ds(gi * _L, _L)]
        for k in range(_L):
            i = gi * _L + k
            tv = jnp.full((_L,), tval[k], jnp.float32)
            for c in range(_DEG // _L):
                tm = tms_v[i, pl.ds(c * _L, _L)]
                nb = nbr_v[i, pl.ds(c * _L, _L)]
                idxp_v[pl.ds(i * _DEG + c * _L, _L)] = jnp.where(
                    tm <= tv, nb, jnp.int32(0))
        return carry

    lax.fori_loop(0, _BPW // _L, mask_body, 0)

    ring = (sem_r0, sem_r1, sem_r2, sem_r3)

    def slot_desc(sl, b):
        return pltpu.make_async_copy(
            feat_hbm.at[idxp_v.at[pl.ds(sl * _G * _DEG, _G * _DEG)]],
            buf_v.at[b], ring[b])

    for p in range(_RING - 1):
        slot_desc(p, p).start()

    def ring_body(so, carry):
        for b in range(_RING):
            sl = so * _RING + b

            @pl.when(sl + _RING - 1 < _NSLOT)
            def _():
                slot_desc(sl + _RING - 1, (b + _RING - 1) % _RING).start()

            slot_desc(sl, b).wait()
            for k in range(_G):
                accs = [buf_v[b, k * _DEG, pl.ds(c * _L, _L)]
                        for c in range(_D // _L)]
                for j in range(1, _DEG):
                    for c in range(_D // _L):
                        accs[c] = accs[c] + buf_v[b, k * _DEG + j,
                                                  pl.ds(c * _L, _L)]
                for c in range(_D // _L):
                    s_v[sl * _G + k, pl.ds(c * _L, _L)] = accs[c]
        return carry

    lax.fori_loop(0, _NSLOT // _RING, ring_body, 0)

    pltpu.sync_copy(s_v, s_out.at[pl.ds(base, _BPW)])
    pltpu.sync_copy(tms_v, tg_out.at[pl.ds(base, _BPW)])
    fs_cp.wait()
    pltpu.sync_copy(fs_v, fs_out.at[pl.ds(base, _BPW)])


def _sc_gather(seed_nodes, t_adj, neighbor_indices, neighbor_times, feat):
    mesh = plsc.VectorSubcoreMesh(core_axis_name="c", subcore_axis_name="s")
    fn = pl.kernel(
        _sc_body,
        out_type=(
            jax.ShapeDtypeStruct((_B, _D), jnp.float32),
            jax.ShapeDtypeStruct((_B, _D), jnp.float32),
            jax.ShapeDtypeStruct((_B, _DEG), jnp.float32),
        ),
        mesh=mesh,
        compiler_params=pltpu.CompilerParams(use_tc_tiling_on_sc=False),
        scratch_types=[
            pltpu.VMEM((_BPW,), jnp.int32),
            pltpu.VMEM((_BPW,), jnp.float32),
            pltpu.VMEM((_BPW, _DEG), jnp.int32),
            pltpu.VMEM((_BPW, _DEG), jnp.float32),
            pltpu.VMEM((_BPW * _DEG,), jnp.int32),
            pltpu.VMEM((_BPW, _D), jnp.float32),
            pltpu.VMEM((_BPW, _D), jnp.float32),
            pltpu.VMEM((_RING, _G * _DEG, _D), jnp.float32),
            pltpu.SemaphoreType.DMA,
            pltpu.SemaphoreType.DMA,
            pltpu.SemaphoreType.DMA,
            pltpu.SemaphoreType.DMA,
            pltpu.SemaphoreType.DMA,
            pltpu.SemaphoreType.DMA,
            pltpu.SemaphoreType.DMA,
        ],
    )
    return fn(seed_nodes, t_adj, neighbor_indices, neighbor_times, feat)


def _tc_head_body(s_ref, fs_ref, tg_ref, ta_ref, f0_ref, w1_ref, w2_ref, o_ref):
    m = (tg_ref[...] <= ta_ref[...][:, None]).astype(jnp.float32)
    dcnt = jnp.sum(m, axis=1)
    pos = dcnt > 0.0
    dinv = jnp.where(pos, lax.rsqrt(jnp.maximum(dcnt, 1e-12)), 0.0)
    s_corr = s_ref[...] - (jnp.float32(_DEG) - dcnt)[:, None] * f0_ref[...]
    agg = jnp.where(pos[:, None],
                    0.5 * fs_ref[...] + 0.5 * dinv[:, None] * s_corr,
                    0.0)
    h = jnp.maximum(
        jnp.dot(agg, w1_ref[...], preferred_element_type=jnp.float32), 0.0)
    o_ref[...] = jnp.dot(h, w2_ref[...], preferred_element_type=jnp.float32)


def _tc_head(s, fs, tg, t_adj, feat0, W1, W2):
    blk = 1024
    ncls = W2.shape[1]
    return pl.pallas_call(
        _tc_head_body,
        grid=(_B // blk,),
        in_specs=[
            pl.BlockSpec((blk, _D), lambda i: (i, 0)),
            pl.BlockSpec((blk, _D), lambda i: (i, 0)),
            pl.BlockSpec((blk, _DEG), lambda i: (i, 0)),
            pl.BlockSpec((blk,), lambda i: (i,)),
            pl.BlockSpec((1, _D), lambda i: (0, 0)),
            pl.BlockSpec((_D, W1.shape[1]), lambda i: (0, 0)),
            pl.BlockSpec((W2.shape[0], ncls), lambda i: (0, 0)),
        ],
        out_specs=pl.BlockSpec((blk, ncls), lambda i: (i, 0)),
        out_shape=jax.ShapeDtypeStruct((_B, ncls), jnp.float32),
    )(s, fs, tg, t_adj, feat0, W1, W2)


def kernel(seed_nodes, t_eval_array, fanouts, delta_t, neighbor_indices,
           neighbor_times, node_features, W1, W2):
    del fanouts
    t_adj = (t_eval_array + delta_t).astype(jnp.float32)
    s, fs, tg = _sc_gather(seed_nodes.astype(jnp.int32), t_adj,
                           neighbor_indices.astype(jnp.int32),
                           neighbor_times, node_features)
    return _tc_head(s, fs, tg, t_adj, node_features[0:1], W1, W2)


# speedup vs baseline: 19.6836x; 1.0037x over previous
import jax
import jax.numpy as jnp
from jax import lax
from jax.experimental import pallas as pl
from jax.experimental.pallas import tpu as pltpu
from jax.experimental.pallas import tpu_sc as plsc

_NC = 2
_NS = 16
_L = 16
_NW = _NC * _NS

_B = 4096
_DEG = 32
_D = 128
_BPW = _B // _NW
_G = 2
_RING = 4
_NSLOT = _BPW // _G


def _sc_body(seed_hbm, tadj_hbm, nbridx_hbm, nbrt_hbm, feat_hbm,
             s_out, fs_out, tg_out,
             seeds_v, tadj_v, nbr_v, tms_v, idxp_v, fs_v, s_v, buf_v,
             sem_a, sem_b, sem_c, sem_r0, sem_r1, sem_r2, sem_r3):
    wid = lax.axis_index("s") * _NC + lax.axis_index("c")
    base = wid * _BPW

    pltpu.sync_copy(seed_hbm.at[pl.ds(base, _BPW)], seeds_v)
    pltpu.sync_copy(tadj_hbm.at[pl.ds(base, _BPW)], tadj_v)

    nbr_cp = pltpu.async_copy(nbridx_hbm.at[seeds_v], nbr_v, sem_a)
    tms_cp = pltpu.async_copy(nbrt_hbm.at[seeds_v], tms_v, sem_b)
    fs_cp = pltpu.async_copy(feat_hbm.at[seeds_v], fs_v, sem_c)
    nbr_cp.wait()
    tms_cp.wait()

    def mask_body(gi, carry):
        tval = tadj_v[pl.ds(gi * _L, _L)]
        for k in range(_L):
            i = gi * _L + k
            tv = jnp.full((_L,), tval[k], jnp.float32)
            for c in range(_DEG // _L):
                tm = tms_v[i, pl.ds(c * _L, _L)]
                nb = nbr_v[i, pl.ds(c * _L, _L)]
                idxp_v[pl.ds(i * _DEG + c * _L, _L)] = jnp.where(
                    tm <= tv, nb, jnp.int32(0))
        return carry

    lax.fori_loop(0, _BPW // _L, mask_body, 0)

    ring = (sem_r0, sem_r1, sem_r2, sem_r3)

    def slot_desc(sl, b):
        return pltpu.make_async_copy(
            feat_hbm.at[idxp_v.at[pl.ds(sl * _G * _DEG, _G * _DEG)]],
            buf_v.at[b], ring[b])

    for p in range(_RING - 1):
        slot_desc(p, p).start()

    def ring_body(so, carry):
        for b in range(_RING):
            sl = so * _RING + b

            @pl.when(sl + _RING - 1 < _NSLOT)
            def _():
                slot_desc(sl + _RING - 1, (b + _RING - 1) % _RING).start()

            slot_desc(sl, b).wait()
            for k in range(_G):
                accs = [buf_v[b, k * _DEG, pl.ds(c * _L, _L)]
                        for c in range(_D // _L)]
                for j in range(1, _DEG):
                    for c in range(_D // _L):
                        accs[c] = accs[c] + buf_v[b, k * _DEG + j,
                                                  pl.ds(c * _L, _L)]
                for c in range(_D // _L):
                    s_v[sl * _G + k, pl.ds(c * _L, _L)] = accs[c]
        return carry

    lax.fori_loop(0, _NSLOT // _RING, ring_body, 0)

    pltpu.sync_copy(s_v, s_out.at[pl.ds(base, _BPW)])
    pltpu.sync_copy(tms_v, tg_out.at[pl.ds(base, _BPW)])
    fs_cp.wait()
    pltpu.sync_copy(fs_v, fs_out.at[pl.ds(base, _BPW)])


def _sc_gather(seed_nodes, t_adj, neighbor_indices, neighbor_times, feat):
    mesh = plsc.VectorSubcoreMesh(core_axis_name="c", subcore_axis_name="s")
    fn = pl.kernel(
        _sc_body,
        out_type=(
            jax.ShapeDtypeStruct((_B, _D), jnp.float32),
            jax.ShapeDtypeStruct((_B, _D), jnp.float32),
            jax.ShapeDtypeStruct((_B, _DEG), jnp.float32),
        ),
        mesh=mesh,
        compiler_params=pltpu.CompilerParams(use_tc_tiling_on_sc=False),
        scratch_types=[
            pltpu.VMEM((_BPW,), jnp.int32),
            pltpu.VMEM((_BPW,), jnp.float32),
            pltpu.VMEM((_BPW, _DEG), jnp.int32),
            pltpu.VMEM((_BPW, _DEG), jnp.float32),
            pltpu.VMEM((_BPW * _DEG,), jnp.int32),
            pltpu.VMEM((_BPW, _D), jnp.float32),
            pltpu.VMEM((_BPW, _D), jnp.float32),
            pltpu.VMEM((_RING, _G * _DEG, _D), jnp.float32),
            pltpu.SemaphoreType.DMA,
            pltpu.SemaphoreType.DMA,
            pltpu.SemaphoreType.DMA,
            pltpu.SemaphoreType.DMA,
            pltpu.SemaphoreType.DMA,
            pltpu.SemaphoreType.DMA,
            pltpu.SemaphoreType.DMA,
        ],
    )
    return fn(seed_nodes, t_adj, neighbor_indices, neighbor_times, feat)


def _tc_head_body(s_ref, fs_ref, tg_ref, ta_ref, f0_ref, w1_ref, w2_ref, o_ref):
    m = (tg_ref[...] <= ta_ref[...][:, None]).astype(jnp.float32)
    dcnt = jnp.sum(m, axis=1)
    pos = dcnt > 0.0
    dinv = jnp.where(pos, lax.rsqrt(jnp.maximum(dcnt, 1e-12)), 0.0)
    s_corr = s_ref[...] - (jnp.float32(_DEG) - dcnt)[:, None] * f0_ref[0:1, :]
    agg = jnp.where(pos[:, None],
                    0.5 * fs_ref[...] + 0.5 * dinv[:, None] * s_corr,
                    0.0)
    h = jnp.maximum(
        jnp.dot(agg, w1_ref[...], preferred_element_type=jnp.float32), 0.0)
    o_ref[...] = jnp.dot(h, w2_ref[...], preferred_element_type=jnp.float32)


def _tc_head(s, fs, tg, t_adj, node_features, W1, W2):
    blk = 1024
    ncls = W2.shape[1]
    return pl.pallas_call(
        _tc_head_body,
        grid=(_B // blk,),
        in_specs=[
            pl.BlockSpec((blk, _D), lambda i: (i, 0)),
            pl.BlockSpec((blk, _D), lambda i: (i, 0)),
            pl.BlockSpec((blk, _DEG), lambda i: (i, 0)),
            pl.BlockSpec((blk,), lambda i: (i,)),
            pl.BlockSpec((8, _D), lambda i: (0, 0)),
            pl.BlockSpec((_D, W1.shape[1]), lambda i: (0, 0)),
            pl.BlockSpec((W2.shape[0], ncls), lambda i: (0, 0)),
        ],
        out_specs=pl.BlockSpec((blk, ncls), lambda i: (i, 0)),
        out_shape=jax.ShapeDtypeStruct((_B, ncls), jnp.float32),
    )(s, fs, tg, t_adj, node_features, W1, W2)


def kernel(seed_nodes, t_eval_array, fanouts, delta_t, neighbor_indices,
           neighbor_times, node_features, W1, W2):
    del fanouts
    t_adj = (t_eval_array + delta_t).astype(jnp.float32)
    s, fs, tg = _sc_gather(seed_nodes.astype(jnp.int32), t_adj,
                           neighbor_indices.astype(jnp.int32),
                           neighbor_times, node_features)
    return _tc_head(s, fs, tg, t_adj, node_features, W1, W2)
